# Initial kernel scaffold; baseline (speedup 1.0000x reference)
import jax
import jax.numpy as jnp
from jax.experimental import pallas as pl


def kernel(x, cell_id, coordinates, nodal_values):
    raise NotImplementedError("write your pallas kernel here")


# speedup vs baseline: 37.9247x; 37.9247x over previous
import functools

import jax
import jax.numpy as jnp
import numpy as np
from jax import lax
from jax.experimental import pallas as pl
from jax.experimental.pallas import tpu as pltpu
from jax.experimental.pallas import tpu_sc as plsc

_CONN = np.array(
    [[1, 2, 7, 6], [2, 3, 8, 7], [3, 4, 9, 8], [4, 5, 10, 9],
     [6, 7, 12, 11], [7, 8, 13, 12], [8, 9, 14, 13], [9, 10, 15, 14],
     [11, 12, 17, 16], [12, 13, 18, 17], [13, 14, 19, 18], [14, 15, 20, 19],
     [16, 17, 22, 21], [17, 18, 23, 22], [18, 19, 24, 23], [19, 20, 25, 24]],
    dtype=np.int32)

_N_PTS = 65536
_N_CELLS = 16
_NODE_PAD = 32

_NC, _NS, _L = 2, 16, 16
_NW = _NC * _NS
_CHUNK = _N_PTS // _NW
_GROUPS = _CHUNK // _L

_IDX = [jnp.asarray(_CONN[:, k] - 1, dtype=jnp.int32) for k in range(4)]

_mesh = plsc.VectorSubcoreMesh(core_axis_name="c", subcore_axis_name="s")


@functools.partial(
    pl.kernel,
    mesh=_mesh,
    out_type=jax.ShapeDtypeStruct((_N_PTS,), jnp.float32),
    compiler_params=pltpu.CompilerParams(needs_layout_passes=False),
    scratch_types=[
        pltpu.VMEM((_CHUNK,), jnp.float32),
        pltpu.VMEM((_CHUNK,), jnp.float32),
        pltpu.VMEM((_CHUNK,), jnp.int32),
        pltpu.VMEM((_CHUNK,), jnp.float32),
        pltpu.VMEM((_NODE_PAD,), jnp.float32),
        pltpu.VMEM((_NODE_PAD,), jnp.float32),
        pltpu.VMEM((_NODE_PAD,), jnp.float32),
        pltpu.VMEM((_N_CELLS,), jnp.int32),
        pltpu.VMEM((_N_CELLS,), jnp.int32),
        pltpu.VMEM((_N_CELLS,), jnp.int32),
        pltpu.VMEM((_N_CELLS,), jnp.int32),
        pltpu.VMEM((_N_CELLS,), jnp.float32),
        pltpu.VMEM((_N_CELLS,), jnp.float32),
        pltpu.VMEM((_N_CELLS,), jnp.float32),
        pltpu.VMEM((_N_CELLS,), jnp.float32),
        pltpu.VMEM((_N_CELLS,), jnp.float32),
        pltpu.VMEM((_N_CELLS,), jnp.float32),
    ],
)
def _sc_interp(x0_hbm, x1_hbm, cid_hbm, cx_hbm, cy_hbm, vv_hbm,
               i0_hbm, i1_hbm, i2_hbm, i3_hbm, out_hbm,
               x0_v, x1_v, cid_v, out_v, cx_v, cy_v, vv_v,
               i0_v, i1_v, i2_v, i3_v,
               c0_v, c1_v, c2_v, c3_v, c4_v, c5_v):
    wid = lax.axis_index("s") * _NC + lax.axis_index("c")
    base = wid * _CHUNK

    pltpu.sync_copy(x0_hbm.at[pl.ds(base, _CHUNK)], x0_v)
    pltpu.sync_copy(x1_hbm.at[pl.ds(base, _CHUNK)], x1_v)
    pltpu.sync_copy(cid_hbm.at[pl.ds(base, _CHUNK)], cid_v)
    pltpu.sync_copy(cx_hbm, cx_v)
    pltpu.sync_copy(cy_hbm, cy_v)
    pltpu.sync_copy(vv_hbm, vv_v)
    pltpu.sync_copy(i0_hbm, i0_v)
    pltpu.sync_copy(i1_hbm, i1_v)
    pltpu.sync_copy(i2_hbm, i2_v)
    pltpu.sync_copy(i3_hbm, i3_v)

    idx = [i0_v[...], i1_v[...], i2_v[...], i3_v[...]]
    nx = [plsc.load_gather(cx_v, [i]) for i in idx]
    ny = [plsc.load_gather(cy_v, [i]) for i in idx]
    nv = [plsc.load_gather(vv_v, [i]) for i in idx]

    c0 = jnp.zeros((_L,), jnp.float32)
    c1 = jnp.zeros((_L,), jnp.float32)
    c2 = jnp.zeros((_L,), jnp.float32)
    c3 = jnp.zeros((_L,), jnp.float32)
    c4 = jnp.zeros((_L,), jnp.float32)
    c5 = jnp.zeros((_L,), jnp.float32)
    for k in range(4):
        ax, ay = nx[k], ny[k]
        bx, by = nx[(k + 1) % 4], ny[(k + 1) % 4]
        ex, ey = nx[(k + 2) % 4], ny[(k + 2) % 4]
        dx, dy = nx[(k + 3) % 4], ny[(k + 3) % 4]
        bb1 = by - ey
        cc1 = ex - bx
        aa1 = -cc1 * by - bb1 * bx
        p12 = cc1 * (ay - by) + bb1 * (ax - bx)
        bb2 = dy - ey
        cc2 = ex - dx
        aa2 = -cc2 * dy - bb2 * dx
        p22 = cc2 * (ay - dy) + bb2 * (ax - dx)
        s = nv[k] / (p12 * p22)
        c0 = c0 + s * aa1 * aa2
        c1 = c1 + s * (aa1 * bb2 + bb1 * aa2)
        c2 = c2 + s * (aa1 * cc2 + cc1 * aa2)
        c3 = c3 + s * bb1 * bb2
        c4 = c4 + s * (bb1 * cc2 + cc1 * bb2)
        c5 = c5 + s * cc1 * cc2
    c0_v[...] = c0
    c1_v[...] = c1
    c2_v[...] = c2
    c3_v[...] = c3
    c4_v[...] = c4
    c5_v[...] = c5

    def step(g, carry):
        off = g * _L
        xx = x0_v[pl.ds(off, _L)]
        yy = x1_v[pl.ds(off, _L)]
        ci = cid_v[pl.ds(off, _L)]
        k0 = plsc.load_gather(c0_v, [ci])
        k1 = plsc.load_gather(c1_v, [ci])
        k2 = plsc.load_gather(c2_v, [ci])
        k3 = plsc.load_gather(c3_v, [ci])
        k4 = plsc.load_gather(c4_v, [ci])
        k5 = plsc.load_gather(c5_v, [ci])
        out_v[pl.ds(off, _L)] = (
            k0 + xx * (k1 + k3 * xx + k4 * yy) + yy * (k2 + k5 * yy))
        return carry

    lax.fori_loop(0, _GROUPS, step, 0, unroll=4)

    pltpu.sync_copy(out_v, out_hbm.at[pl.ds(base, _CHUNK)])


def kernel(x, cell_id, coordinates, nodal_values):
    x0 = x[:, 0]
    x1 = x[:, 1]
    cid = cell_id.astype(jnp.int32)
    coords = coordinates.reshape(-1, 2)
    cx = jnp.zeros((_NODE_PAD,), jnp.float32).at[:coords.shape[0]].set(coords[:, 0])
    cy = jnp.zeros((_NODE_PAD,), jnp.float32).at[:coords.shape[0]].set(coords[:, 1])
    vv = jnp.zeros((_NODE_PAD,), jnp.float32).at[:coords.shape[0]].set(
        nodal_values.reshape(-1))
    return _sc_interp(x0, x1, cid, cx, cy, vv, _IDX[0], _IDX[1], _IDX[2], _IDX[3])
